# Initial kernel scaffold; baseline (speedup 1.0000x reference)
import jax
import jax.numpy as jnp
from jax.experimental import pallas as pl


def kernel(x, edge_index, batch, Wl, bl, Wr, W1, b1, W2, b2):
    raise NotImplementedError("write your pallas kernel here")


# speedup vs baseline: 5.3375x; 5.3375x over previous
import functools

import jax
import jax.numpy as jnp
from jax import lax
from jax.experimental import pallas as pl
from jax.experimental.pallas import tpu as pltpu
from jax.experimental.pallas import tpu_sc as plsc

N = 10000
D = 128
HD = D // 2
MSG = 32
NDEG = 11
E = 320000

NC, NS, K = 2, 16, 128
NW = NC * NS
CH = 158
EPT = CH * K
NPAD = 10112
RPT = NPAD // NS

_sc_mesh = plsc.VectorSubcoreMesh(core_axis_name="c", subcore_axis_name="s")


@functools.partial(
    pl.kernel,
    out_type=(
        jax.ShapeDtypeStruct((NC * NPAD, HD), jnp.float32),
        jax.ShapeDtypeStruct((NPAD, 8), jnp.float32),
    ),
    mesh=_sc_mesh,
    compiler_params=pltpu.CompilerParams(use_tc_tiling_on_sc=False),
    scratch_types=[
        pltpu.VMEM((CH, K), jnp.int32),
        pltpu.VMEM((CH, K), jnp.int32),
        pltpu.VMEM((K, HD), jnp.float32),
        pltpu.VMEM((K, HD), jnp.float32),
        pltpu.VMEM((K, 8), jnp.float32),
        pltpu.VMEM_SHARED((NPAD, HD), jnp.float32),
        pltpu.VMEM_SHARED((NPAD, 8), jnp.float32),
        pltpu.SemaphoreType.DMA,
        pltpu.SemaphoreType.DMA,
    ],
)
def _sc_segment_sum(x_hbm, src_hbm, dst_hbm, zh_hbm, zd_hbm, oh_hbm,
                    ph_hbm, pd_hbm,
                    src_v, dst_v, buf0, buf1, ones_v, ha, dacc, sem0, sem1):
    c = lax.axis_index("c")
    s = lax.axis_index("s")
    wid = c * NS + s
    r0 = s * RPT
    is_c0 = c == 0

    pltpu.sync_copy(zh_hbm, ha.at[pl.ds(r0, RPT)])

    @pl.when(is_c0)
    def _():
        pltpu.sync_copy(zd_hbm, dacc.at[pl.ds(r0, RPT)])
        pltpu.sync_copy(oh_hbm, ones_v)

    pltpu.sync_copy(src_hbm.at[wid], src_v)
    pltpu.sync_copy(dst_hbm.at[wid], dst_v)
    plsc.subcore_barrier()

    pltpu.async_copy(x_hbm.at[src_v.at[0]], buf0, sem0)

    def step(t, carry):
        j0 = t * 2
        j1 = j0 + 1
        pltpu.async_copy(x_hbm.at[src_v.at[j1]], buf1, sem1)
        pltpu.make_async_copy(x_hbm.at[src_v.at[j0]], buf0, sem0).wait()
        pltpu.sync_copy(buf0, ha.at[dst_v.at[j0]], add=True)

        @pl.when(is_c0)
        def _():
            pltpu.sync_copy(ones_v, dacc.at[dst_v.at[j0]], add=True)

        @pl.when(j0 + 2 < CH)
        def _():
            pltpu.async_copy(x_hbm.at[src_v.at[j0 + 2]], buf0, sem0)

        pltpu.make_async_copy(x_hbm.at[src_v.at[j1]], buf1, sem1).wait()
        pltpu.sync_copy(buf1, ha.at[dst_v.at[j1]], add=True)

        @pl.when(is_c0)
        def _():
            pltpu.sync_copy(ones_v, dacc.at[dst_v.at[j1]], add=True)

        return carry

    lax.fori_loop(0, CH // 2, step, 0)
    plsc.subcore_barrier()

    pltpu.sync_copy(ha.at[pl.ds(r0, RPT)], ph_hbm.at[pl.ds(c * NPAD + r0, RPT)])

    @pl.when(is_c0)
    def _():
        pltpu.sync_copy(dacc.at[pl.ds(r0, RPT)], pd_hbm.at[pl.ds(r0, RPT)])


GB = 8
BR = NPAD // GB


def _tc_body(ph_ref, pd_ref, x_ref, ah_ref, b_ref, bc_ref, w1_ref, b1_ref,
             w2_ref, b2_ref, emb_ref, out_ref):
    xb = x_ref[...]
    r = (jnp.dot(ph_ref[0], ah_ref[0], preferred_element_type=jnp.float32)
         + jnp.dot(ph_ref[1], ah_ref[1], preferred_element_type=jnp.float32)
         + jnp.dot(xb, b_ref[...], preferred_element_type=jnp.float32)
         + bc_ref[...])
    deg = jnp.minimum(pd_ref[:, 0:1], float(NDEG - 1))
    conv = jnp.zeros((BR, MSG), jnp.float32)
    for i in range(NDEG):
        conv = jnp.where(deg == float(i), r[:, i * MSG:(i + 1) * MSG], conv)
    emb_ref[...] = conv
    t = jnp.maximum(conv, 0.0)
    t = jnp.dot(t, w1_ref[...], preferred_element_type=jnp.float32) + b1_ref[...]
    out_ref[...] = (jnp.dot(t, w2_ref[...], preferred_element_type=jnp.float32)
                    + b2_ref[...])


_tc_dense = pl.pallas_call(
    _tc_body,
    grid=(GB,),
    in_specs=[
        pl.BlockSpec((NC, BR, HD), lambda g: (0, g, 0)),
        pl.BlockSpec((BR, 8), lambda g: (g, 0)),
        pl.BlockSpec((BR, D), lambda g: (g, 0)),
        pl.BlockSpec((NC, HD, NDEG * MSG), lambda g: (0, 0, 0)),
        pl.BlockSpec((D, NDEG * MSG), lambda g: (0, 0)),
        pl.BlockSpec((NDEG * MSG,), lambda g: (0,)),
        pl.BlockSpec((MSG, MSG), lambda g: (0, 0)),
        pl.BlockSpec((MSG,), lambda g: (0,)),
        pl.BlockSpec((MSG, MSG), lambda g: (0, 0)),
        pl.BlockSpec((MSG,), lambda g: (0,)),
    ],
    out_specs=[
        pl.BlockSpec((BR, MSG), lambda g: (g, 0)),
        pl.BlockSpec((BR, MSG), lambda g: (g, 0)),
    ],
    out_shape=[
        jax.ShapeDtypeStruct((NPAD, MSG), jnp.float32),
        jax.ShapeDtypeStruct((NPAD, MSG), jnp.float32),
    ],
)


def kernel(x, edge_index, batch, Wl, bl, Wr, W1, b1, W2, b2):
    src = edge_index[0]
    dst = edge_index[1]
    pad = NS * EPT - E
    srcp = jnp.concatenate([src, jnp.zeros((pad,), jnp.int32)])
    dstp = jnp.concatenate([dst, jnp.full((pad,), N, jnp.int32)])
    src2 = srcp.reshape(NS, CH, K)
    dst2 = dstp.reshape(NS, CH, K)
    src4 = jnp.concatenate([src2, src2 + N], axis=0)
    dst4 = jnp.concatenate([dst2, dst2], axis=0)
    xcat = jnp.concatenate([x[:, :HD], x[:, HD:]], axis=0)
    zh = jnp.zeros((RPT, HD), jnp.float32)
    zd = jnp.zeros((RPT, 8), jnp.float32)
    oh = jnp.ones((K, 8), jnp.float32)

    ph, pd = _sc_segment_sum(xcat, src4, dst4, zh, zd, oh)

    x_pad = jnp.concatenate([x, jnp.zeros((NPAD - N, D), jnp.float32)])
    a = jnp.transpose(Wl, (2, 0, 1)).reshape(D, NDEG * MSG)
    ah = jnp.stack([a[:HD], a[HD:]])
    b = jnp.transpose(Wr, (2, 0, 1)).reshape(D, NDEG * MSG)
    bc = bl.reshape(NDEG * MSG)

    emb, out = _tc_dense(ph.reshape(NC, NPAD, HD), pd, x_pad, ah, b, bc,
                         W1.T, b1, W2.T, b2)
    return emb[:N], out[:N]
